# baseline (speedup 1.0000x reference)
import functools

import numpy as np

import jax
import jax.numpy as jnp
from jax import lax
from jax.experimental import pallas as pl
from jax.experimental.pallas import tpu as pltpu
from jax.experimental.pallas import tpu_sc as plsc

N_NODES = 10000
N_EDGES = 320000
D_FEAT = 128
D_HALF = 64
D_EDGE = 16

NW = 32
EPW = N_EDGES // NW
CH = 80
NCH = EPW // CH
N_PAD = 10240
NPT = N_PAD // 16


def _rsqrt16(a):
    i = plsc.bitcast(a, jnp.int32)
    i = jnp.int32(0x5F3759DF) - (i >> 1)
    y = plsc.bitcast(i, jnp.float32)
    for _ in range(3):
        y = y * (jnp.float32(1.5) - jnp.float32(0.5) * a * y * y)
    return y


NBUF = 5


def _sc_aggregate_body(x0_hbm, x1_hbm, sqn_hbm, row_hbm, col_hbm, ea_hbm,
                       sqna_hbm,
                       aggae_out, aggx0_out, aggx1_out,
                       sqn_v, row_v, col_v, sqna_v, ea_v, gbufs, bufs, w_v,
                       agg_sh, gsem, ssem):
    cid = lax.axis_index("c")
    sid = lax.axis_index("s")
    wid = cid * 16 + sid

    pltpu.sync_copy(sqn_hbm, sqn_v)
    pltpu.sync_copy(row_hbm.at[wid], row_v)
    pltpu.sync_copy(col_hbm.at[wid], col_v)

    zeros16 = jnp.zeros((16,), jnp.float32)
    onehot = jnp.where(lax.iota(jnp.int32, 16) == 0,
                       jnp.float32(1.0), jnp.float32(0.0))
    zbase = sid * NPT

    def wait_scatter(j):
        pltpu.make_async_copy(bufs.at[0], agg_sh.at[col_v.at[j]],
                              ssem).wait()

    for p, (xt_hbm, out_hbm) in enumerate(
            ((None, aggae_out), (x0_hbm, aggx0_out), (x1_hbm, aggx1_out))):
        def zero_row(i, c):
            for d in range(4):
                bufs[0, i, pl.ds(d * 16, 16)] = zeros16
            return c

        lax.fori_loop(0, CH, zero_row, 0)
        for t in range(NPT // CH):
            pltpu.sync_copy(bufs.at[0],
                            agg_sh.at[pl.ds(zbase + t * CH, CH)])
        plsc.subcore_barrier()

        if p > 0:
            pltpu.async_copy(xt_hbm.at[row_v.at[0]], gbufs.at[0], gsem)
            pltpu.async_copy(xt_hbm.at[row_v.at[1]], gbufs.at[1], gsem)

        def outer(j5, c, p=p, xt_hbm=xt_hbm):
            for r in range(NBUF):
                j = j5 * NBUF + r
                buf = bufs.at[r]
                gbuf = gbufs.at[r]
                pl.when(j >= 3)(lambda: wait_scatter(j - 3))
                if p > 0:
                    def _ahead(j=j, r=r, xt_hbm=xt_hbm):
                        pltpu.async_copy(xt_hbm.at[row_v.at[j + 2]],
                                         gbufs.at[(r + 2) % NBUF], gsem)
                    pl.when(j + 2 < NCH)(_ahead)
                    pltpu.make_async_copy(xt_hbm.at[row_v.at[j]], gbuf,
                                          gsem).wait()
                else:
                    pltpu.sync_copy(ea_hbm.at[wid, j], ea_v)
                    pltpu.sync_copy(sqna_hbm.at[wid, j], sqna_v)
                    for k in range(CH // 16):
                        ridx = row_v[j, pl.ds(k * 16, 16)]
                        sq = (plsc.load_gather(sqn_v, [ridx])
                              + sqna_v[pl.ds(k * 16, 16)])
                        y = _rsqrt16(jnp.maximum(sq, jnp.float32(1e-30)))
                        w = jnp.where(sq > jnp.float32(1e-24), y,
                                      jnp.float32(1e12))
                        w_v[pl.ds(j * CH + k * 16, 16)] = w

                def per_edge(i, cc, buf=buf, gbuf=gbuf, p=p):
                    w = w_v[pl.ds(j * CH + i, 16)][0]
                    if p == 0:
                        buf[i, pl.ds(0, 16)] = ea_v[i, :] * w
                        buf[i, pl.ds(16, 16)] = onehot
                        buf[i, pl.ds(32, 16)] = zeros16
                        buf[i, pl.ds(48, 16)] = zeros16
                    else:
                        for d in range(2):
                            vi = gbuf[i, pl.ds(d * 16, 16)]
                            lo = plsc.bitcast(vi << 16, jnp.float32)
                            hi = plsc.bitcast(
                                vi & jnp.int32(-65536), jnp.float32)
                            buf[i, pl.ds(d * 32, 16)] = lo * w
                            buf[i, pl.ds(d * 32 + 16, 16)] = hi * w
                    return cc

                lax.fori_loop(0, CH, per_edge, 0, unroll=2)
                pltpu.async_copy(buf, agg_sh.at[col_v.at[j]], ssem, add=True)
            return c

        lax.fori_loop(0, NCH // NBUF, outer, 0)
        for jt in (NCH - 3, NCH - 2, NCH - 1):
            wait_scatter(jt)
        plsc.subcore_barrier()

        pltpu.sync_copy(agg_sh.at[pl.ds(zbase, NPT)],
                        out_hbm.at[cid, pl.ds(zbase, NPT)])


_SC_MESH = plsc.VectorSubcoreMesh(core_axis_name="c", subcore_axis_name="s")

_sc_aggregate = pl.kernel(
    _sc_aggregate_body,
    out_type=(
        jax.ShapeDtypeStruct((2, N_PAD, D_HALF), jnp.float32),
        jax.ShapeDtypeStruct((2, N_PAD, D_HALF), jnp.float32),
        jax.ShapeDtypeStruct((2, N_PAD, D_HALF), jnp.float32),
    ),
    mesh=_SC_MESH,
    scratch_types=[
        pltpu.VMEM((N_NODES,), jnp.float32),
        pltpu.VMEM((NCH, CH), jnp.int32),
        pltpu.VMEM((NCH, CH), jnp.int32),
        pltpu.VMEM((CH,), jnp.float32),
        pltpu.VMEM((CH, D_EDGE), jnp.float32),
        pltpu.VMEM((NBUF, CH, D_HALF // 2), jnp.int32),
        pltpu.VMEM((NBUF, CH, D_HALF), jnp.float32),
        pltpu.VMEM((EPW + 16,), jnp.float32),
        pltpu.VMEM_SHARED((N_PAD, D_HALF), jnp.float32),
        pltpu.SemaphoreType.DMA,
        pltpu.SemaphoreType.DMA,
    ],
    compiler_params=pltpu.CompilerParams(needs_layout_passes=False,
                                         use_tc_tiling_on_sc=False),
    name="sage_sc_aggregate",
)


def _sq_body(x_ref, o_ref):
    v = x_ref[...]
    o_ref[...] = jnp.sum(v * v, axis=1, keepdims=True)


def _row_sqnorm(x, blk):
    rows, d = x.shape
    return pl.pallas_call(
        _sq_body,
        grid=(rows // blk,),
        in_specs=[pl.BlockSpec((blk, d), lambda i: (i, 0))],
        out_specs=pl.BlockSpec((blk, 1), lambda i: (i, 0)),
        out_shape=jax.ShapeDtypeStruct((rows, 1), jnp.float32),
    )(x)


def _post_body(ae_ref, ax0_ref, ax1_ref, wx0_ref, wx1_ref, we_ref, b_ref,
               *out_refs, act, split_out):
    ae = ae_ref[0] + ae_ref[1]
    ax0 = ax0_ref[0] + ax0_ref[1]
    ax1 = ax1_ref[0] + ax1_ref[1]
    cnt = ae[:, 16:17]
    inv = jnp.float32(1.0) / jnp.maximum(cnt, jnp.float32(1.0))
    h = (lax.dot(ax0, wx0_ref[...], preferred_element_type=jnp.float32)
         + lax.dot(ax1, wx1_ref[...], preferred_element_type=jnp.float32)
         + lax.dot(ae[:, :16], we_ref[...], preferred_element_type=jnp.float32))
    h = h * inv + b_ref[...]
    n = jnp.sqrt(jnp.sum(h * h, axis=1, keepdims=True))
    h = act(h / jnp.maximum(n, jnp.float32(1e-12)))
    if split_out:
        o0_ref, o1_ref, sq_ref = out_refs
        o0_ref[...] = h[:, :D_HALF].astype(jnp.bfloat16)
        o1_ref[...] = h[:, D_HALF:].astype(jnp.bfloat16)
        sq_ref[...] = jnp.sum(h * h, axis=1, keepdims=True)
    else:
        out_refs[0][...] = h


def _post(ae, ax0, ax1, wx0, wx1, we, b, act, split_out, blk=1000):
    body = functools.partial(_post_body, act=act, split_out=split_out)
    if split_out:
        out_specs = [
            pl.BlockSpec((blk, D_HALF), lambda i: (i, 0)),
            pl.BlockSpec((blk, D_HALF), lambda i: (i, 0)),
            pl.BlockSpec((blk, 1), lambda i: (i, 0)),
        ]
        out_shape = [
            jax.ShapeDtypeStruct((N_NODES, D_HALF), jnp.bfloat16),
            jax.ShapeDtypeStruct((N_NODES, D_HALF), jnp.bfloat16),
            jax.ShapeDtypeStruct((N_NODES, 1), jnp.float32),
        ]
    else:
        out_specs = [pl.BlockSpec((blk, D_FEAT), lambda i: (i, 0))]
        out_shape = [jax.ShapeDtypeStruct((N_NODES, D_FEAT), jnp.float32)]
    return pl.pallas_call(
        body,
        grid=(N_NODES // blk,),
        in_specs=[
            pl.BlockSpec((2, blk, D_HALF), lambda i: (0, i, 0)),
            pl.BlockSpec((2, blk, D_HALF), lambda i: (0, i, 0)),
            pl.BlockSpec((2, blk, D_HALF), lambda i: (0, i, 0)),
            pl.BlockSpec((D_HALF, D_FEAT), lambda i: (0, 0)),
            pl.BlockSpec((D_HALF, D_FEAT), lambda i: (0, 0)),
            pl.BlockSpec((D_EDGE, D_FEAT), lambda i: (0, 0)),
            pl.BlockSpec((1, D_FEAT), lambda i: (0, 0)),
        ],
        out_specs=out_specs,
        out_shape=out_shape,
    )(ae, ax0, ax1, wx0, wx1, we, b)


_PERM = np.concatenate([
    np.concatenate([np.arange(d * 32, (d + 1) * 32, 2),
                    np.arange(d * 32 + 1, (d + 1) * 32, 2)])
    for d in range(D_HALF // 32)
])


def _layer(x0, x1, sqn, row, col, ea_r, sqna, W, b, act, split_out):
    ae, ax0, ax1 = _sc_aggregate(x0, x1, sqn, row, col, ea_r, sqna)
    return _post(ae, ax0, ax1,
                 W[:D_HALF][_PERM], W[D_HALF:D_FEAT][_PERM], W[D_FEAT:],
                 b.reshape(1, D_FEAT), act, split_out)


def kernel(x, edge_index, edge_attr, W1, b1, W2, b2):
    ei = edge_index.astype(jnp.int32)
    row = ei[0].reshape(NW, NCH, CH)
    col = ei[1].reshape(NW, NCH, CH)
    ea_r = edge_attr.reshape(NW, NCH, CH, D_EDGE)

    sqna = _row_sqnorm(edge_attr, blk=8000).reshape(NW, NCH, CH)
    sqnx = _row_sqnorm(x, blk=1000).reshape(N_NODES)

    def _pack(xh):
        return lax.bitcast_convert_type(
            xh.reshape(N_NODES, D_HALF // 2, 2), jnp.int32)

    h0, h1, sqh1 = _layer(_pack(x[:, :D_HALF].astype(jnp.bfloat16)),
                          _pack(x[:, D_HALF:].astype(jnp.bfloat16)), sqnx,
                          row, col, ea_r, sqna, W1, b1,
                          lambda h: jnp.maximum(h, 0.0), split_out=True)
    (out,) = _layer(_pack(h0), _pack(h1), sqh1.reshape(N_NODES),
                    row, col, ea_r, sqna,
                    W2, b2, lambda h: jax.nn.sigmoid(h), split_out=False)
    return out


# speedup vs baseline: 1.0210x; 1.0210x over previous
import functools

import jax
import jax.numpy as jnp
from jax import lax
from jax.experimental import pallas as pl
from jax.experimental.pallas import tpu as pltpu
from jax.experimental.pallas import tpu_sc as plsc

N_NODES = 10000
N_EDGES = 320000
D_FEAT = 128
D_HALF = 64
D_EDGE = 16
BW = 80

NW = 32
EPW = N_EDGES // NW
CH = 80
NCH = EPW // CH
N_PAD = 10240
NPT = N_PAD // 16
NBUF = 5


def _rsqrt16(a):
    i = plsc.bitcast(a, jnp.int32)
    i = jnp.int32(0x5F3759DF) - (i >> 1)
    y = plsc.bitcast(i, jnp.float32)
    for _ in range(3):
        y = y * (jnp.float32(1.5) - jnp.float32(0.5) * a * y * y)
    return y


def _sc_aggregate_body(x0_hbm, x1_hbm, sqn_hbm, rc_hbm, ea_hbm, sqna_hbm,
                       aggA_out, aggB_out,
                       sqn_v, rc_v, row_r, col_r, sqna_r, ea_r2, gbufs, bufs,
                       w_c, agg_sh, gsem, ssem, qsem, esem):
    cid = lax.axis_index("c")
    sid = lax.axis_index("s")
    wid = cid * 16 + sid

    pltpu.sync_copy(sqn_hbm, sqn_v)
    pltpu.sync_copy(rc_hbm.at[wid], rc_v)

    zeros16 = jnp.zeros((16,), jnp.float32)
    onehot = jnp.where(lax.iota(jnp.int32, 16) == 0,
                       jnp.float32(1.0), jnp.float32(0.0))
    zbase = sid * NPT

    def unpack_row(j, s):
        for k in range(CH // 16):
            rc = rc_v[j, pl.ds(k * 16, 16)]
            row_r[s, pl.ds(k * 16, 16)] = rc & jnp.int32(0xFFFF)

    def unpack_col(j, s):
        for k in range(CH // 16):
            rc = rc_v[j, pl.ds(k * 16, 16)]
            col_r[s, pl.ds(k * 16, 16)] = rc >> 16

    def wait_scatter():
        pltpu.make_async_copy(bufs.at[0], agg_sh.at[col_r.at[0]],
                              ssem).wait()

    for p, (xt_hbm, out_hbm) in enumerate(
            ((x0_hbm, aggA_out), (x1_hbm, aggB_out))):
        def zero_row(i, c):
            for d in range(BW // 16):
                bufs[0, i, pl.ds(d * 16, 16)] = zeros16
            return c

        lax.fori_loop(0, CH, zero_row, 0)
        for t in range(NPT // CH):
            pltpu.sync_copy(bufs.at[0],
                            agg_sh.at[pl.ds(zbase + t * CH, CH)])
        plsc.subcore_barrier()

        unpack_row(0, 0)
        unpack_row(1, 1)
        pltpu.async_copy(xt_hbm.at[row_r.at[0]], gbufs.at[0], gsem)
        pltpu.async_copy(xt_hbm.at[row_r.at[1]], gbufs.at[1], gsem)
        pltpu.sync_copy(sqna_hbm.at[wid, 0], sqna_r.at[0])
        if p == 0:
            pltpu.sync_copy(ea_hbm.at[wid, 0], ea_r2.at[0])

        def outer(j5, c, p=p, xt_hbm=xt_hbm):
            for r in range(NBUF):
                j = j5 * NBUF + r
                buf = bufs.at[r]
                gbuf = gbufs.at[r]
                pl.when(j >= 3)(lambda: wait_scatter())

                def _ahead(j=j, r=r, xt_hbm=xt_hbm):
                    unpack_row(j + 2, (r + 2) % NBUF)
                    pltpu.async_copy(xt_hbm.at[row_r.at[(r + 2) % NBUF]],
                                     gbufs.at[(r + 2) % NBUF], gsem)
                pl.when(j + 2 < NCH)(_ahead)

                def _next_meta(j=j, p=p):
                    pltpu.async_copy(sqna_hbm.at[wid, j + 1],
                                     sqna_r.at[(j + 1) % 2], qsem)
                    if p == 0:
                        pltpu.async_copy(ea_hbm.at[wid, j + 1],
                                         ea_r2.at[(j + 1) % 2], esem)
                pl.when(j + 1 < NCH)(_next_meta)

                def _wait_meta(p=p):
                    pltpu.make_async_copy(sqna_hbm.at[wid, 0],
                                          sqna_r.at[0], qsem).wait()
                    if p == 0:
                        pltpu.make_async_copy(ea_hbm.at[wid, 0],
                                              ea_r2.at[0], esem).wait()
                pl.when(j >= 1)(_wait_meta)

                unpack_col(j, r)
                sqna_c = sqna_r.at[j % 2]
                for k in range(CH // 16):
                    ridx = row_r[r, pl.ds(k * 16, 16)]
                    pair = plsc.load_gather(sqn_v, [ridx >> 1])
                    sqx = jnp.where(
                        (ridx & jnp.int32(1)) == 0,
                        plsc.bitcast(pair << 16, jnp.float32),
                        plsc.bitcast(pair & jnp.int32(-65536), jnp.float32))
                    sq = sqx + sqna_c[pl.ds(k * 16, 16)]
                    y = _rsqrt16(jnp.maximum(sq, jnp.float32(1e-30)))
                    w = jnp.where(sq > jnp.float32(1e-24), y,
                                  jnp.float32(1e12))
                    w_c[pl.ds(k * 16, 16)] = w
                pltpu.make_async_copy(xt_hbm.at[row_r.at[r]], gbuf,
                                      gsem).wait()

                ea_c = ea_r2.at[j % 2]

                def per_edge(i, cc, buf=buf, gbuf=gbuf, ea_c=ea_c, p=p):
                    w = w_c[pl.ds(i, 16)][0]
                    for d in range(D_HALF // 16):
                        buf[i, pl.ds(d * 16, 16)] = \
                            gbuf[i, pl.ds(d * 16, 16)] * w
                    if p == 0:
                        buf[i, pl.ds(D_HALF, 16)] = ea_c[i, :] * w
                    else:
                        buf[i, pl.ds(D_HALF, 16)] = onehot
                    return cc

                lax.fori_loop(0, CH, per_edge, 0, unroll=2)
                pltpu.async_copy(buf, agg_sh.at[col_r.at[r]], ssem, add=True)
            return c

        lax.fori_loop(0, NCH // NBUF, outer, 0)
        for _ in range(3):
            wait_scatter()
        plsc.subcore_barrier()

        pltpu.sync_copy(agg_sh.at[pl.ds(zbase, NPT)],
                        out_hbm.at[cid, pl.ds(zbase, NPT)])


_SC_MESH = plsc.VectorSubcoreMesh(core_axis_name="c", subcore_axis_name="s")

_sc_aggregate = pl.kernel(
    _sc_aggregate_body,
    out_type=(
        jax.ShapeDtypeStruct((2, N_PAD, BW), jnp.float32),
        jax.ShapeDtypeStruct((2, N_PAD, BW), jnp.float32),
    ),
    mesh=_SC_MESH,
    scratch_types=[
        pltpu.VMEM((N_NODES // 2,), jnp.int32),
        pltpu.VMEM((NCH, CH), jnp.int32),
        pltpu.VMEM((NBUF, CH), jnp.int32),
        pltpu.VMEM((NBUF, CH), jnp.int32),
        pltpu.VMEM((2, CH), jnp.float32),
        pltpu.VMEM((2, CH, D_EDGE), jnp.float32),
        pltpu.VMEM((NBUF, CH, D_HALF), jnp.float32),
        pltpu.VMEM((NBUF, CH, BW), jnp.float32),
        pltpu.VMEM((CH + 16,), jnp.float32),
        pltpu.VMEM_SHARED((N_PAD, BW), jnp.float32),
        pltpu.SemaphoreType.DMA,
        pltpu.SemaphoreType.DMA,
        pltpu.SemaphoreType.DMA,
        pltpu.SemaphoreType.DMA,
    ],
    compiler_params=pltpu.CompilerParams(needs_layout_passes=False,
                                         use_tc_tiling_on_sc=False),
    name="sage_sc_aggregate",
)


def _sq_body(x_ref, o_ref):
    v = x_ref[...]
    o_ref[...] = jnp.sum(v * v, axis=1, keepdims=True)


def _row_sqnorm(x, blk):
    rows, d = x.shape
    return pl.pallas_call(
        _sq_body,
        grid=(rows // blk,),
        in_specs=[pl.BlockSpec((blk, d), lambda i: (i, 0))],
        out_specs=pl.BlockSpec((blk, 1), lambda i: (i, 0)),
        out_shape=jax.ShapeDtypeStruct((rows, 1), jnp.float32),
    )(x)


def _post_body(aggA_ref, aggB_ref, wx0_ref, wx1_ref, we_ref, b_ref,
               *out_refs, act, split_out):
    a = aggA_ref[0] + aggA_ref[1]
    bb = aggB_ref[0] + aggB_ref[1]
    cnt = bb[:, D_HALF:D_HALF + 1]
    inv = jnp.float32(1.0) / jnp.maximum(cnt, jnp.float32(1.0))
    h = (lax.dot(a[:, :D_HALF], wx0_ref[...],
                 preferred_element_type=jnp.float32)
         + lax.dot(bb[:, :D_HALF], wx1_ref[...],
                   preferred_element_type=jnp.float32)
         + lax.dot(a[:, D_HALF:], we_ref[...],
                   preferred_element_type=jnp.float32))
    h = h * inv + b_ref[...]
    n = jnp.sqrt(jnp.sum(h * h, axis=1, keepdims=True))
    h = act(h / jnp.maximum(n, jnp.float32(1e-12)))
    if split_out:
        o0_ref, o1_ref, sq_ref = out_refs
        o0_ref[...] = h[:, :D_HALF]
        o1_ref[...] = h[:, D_HALF:]
        sq_ref[...] = jnp.sum(h * h, axis=1, keepdims=True)
    else:
        out_refs[0][...] = h


def _post(aggA, aggB, wx0, wx1, we, b, act, split_out, blk=1000):
    body = functools.partial(_post_body, act=act, split_out=split_out)
    if split_out:
        out_specs = [
            pl.BlockSpec((blk, D_HALF), lambda i: (i, 0)),
            pl.BlockSpec((blk, D_HALF), lambda i: (i, 0)),
            pl.BlockSpec((blk, 1), lambda i: (i, 0)),
        ]
        out_shape = [
            jax.ShapeDtypeStruct((N_NODES, D_HALF), jnp.float32),
            jax.ShapeDtypeStruct((N_NODES, D_HALF), jnp.float32),
            jax.ShapeDtypeStruct((N_NODES, 1), jnp.float32),
        ]
    else:
        out_specs = [pl.BlockSpec((blk, D_FEAT), lambda i: (i, 0))]
        out_shape = [jax.ShapeDtypeStruct((N_NODES, D_FEAT), jnp.float32)]
    return pl.pallas_call(
        body,
        grid=(N_NODES // blk,),
        in_specs=[
            pl.BlockSpec((2, blk, BW), lambda i: (0, i, 0)),
            pl.BlockSpec((2, blk, BW), lambda i: (0, i, 0)),
            pl.BlockSpec((D_HALF, D_FEAT), lambda i: (0, 0)),
            pl.BlockSpec((D_HALF, D_FEAT), lambda i: (0, 0)),
            pl.BlockSpec((D_EDGE, D_FEAT), lambda i: (0, 0)),
            pl.BlockSpec((1, D_FEAT), lambda i: (0, 0)),
        ],
        out_specs=out_specs,
        out_shape=out_shape,
    )(aggA, aggB, wx0, wx1, we, b)


def _layer(x0, x1, sqn, rc, ea_r, sqna, W, b, act, split_out):
    aggA, aggB = _sc_aggregate(x0, x1, sqn, rc, ea_r, sqna)
    return _post(aggA, aggB,
                 W[:D_HALF], W[D_HALF:D_FEAT], W[D_FEAT:],
                 b.reshape(1, D_FEAT), act, split_out)


def kernel(x, edge_index, edge_attr, W1, b1, W2, b2):
    ei = edge_index.astype(jnp.int32)
    rc = (ei[0] | (ei[1] << 16)).reshape(NW, NCH, CH)
    ea_r = edge_attr.reshape(NW, NCH, CH, D_EDGE)

    sqna = _row_sqnorm(edge_attr, blk=8000).reshape(NW, NCH, CH)
    sqnx = _row_sqnorm(x, blk=1000).reshape(N_NODES)

    def _packn(s):
        return lax.bitcast_convert_type(
            s.astype(jnp.bfloat16).reshape(N_NODES // 2, 2), jnp.int32)

    h0, h1, sqh1 = _layer(x[:, :D_HALF], x[:, D_HALF:], _packn(sqnx),
                          rc, ea_r, sqna, W1, b1,
                          lambda h: jnp.maximum(h, 0.0), split_out=True)
    (out,) = _layer(h0, h1, _packn(sqh1.reshape(N_NODES)), rc, ea_r, sqna,
                    W2, b2, lambda h: jax.nn.sigmoid(h), split_out=False)
    return out


# speedup vs baseline: 1.3495x; 1.3217x over previous
import functools

import jax
import jax.numpy as jnp
from jax import lax
from jax.experimental import pallas as pl
from jax.experimental.pallas import tpu as pltpu
from jax.experimental.pallas import tpu_sc as plsc

N_NODES = 10000
N_EDGES = 320000
D_FEAT = 128
D_HALF = 64
D_EDGE = 16

NW = 32
EPW = N_EDGES // NW
CH = 80
NCH = EPW // CH
N_PAD = 10240
NPT = N_PAD // 16
NBUF = 5


def _rsqrt16(a):
    i = plsc.bitcast(a, jnp.int32)
    i = jnp.int32(0x5F3759DF) - (i >> 1)
    y = plsc.bitcast(i, jnp.float32)
    for _ in range(3):
        y = y * (jnp.float32(1.5) - jnp.float32(0.5) * a * y * y)
    return y


def _sc_aggregate_body(x0_hbm, x1_hbm, sqn_hbm, row_hbm, col_hbm, ea_hbm,
                       sqna_hbm,
                       aggae_out, aggx0_out, aggx1_out,
                       sqn_v, row_v, col_v, sqna_v, ea_v, bufs, w_v, zx_v,
                       agg_sh, gsem, ssem):
    cid = lax.axis_index("c")
    sid = lax.axis_index("s")
    wid = cid * 16 + sid

    pltpu.sync_copy(sqn_hbm, sqn_v)
    pltpu.sync_copy(row_hbm.at[wid], row_v)
    pltpu.sync_copy(col_hbm.at[wid], col_v)
    pltpu.sync_copy(sqna_hbm.at[wid], sqna_v)

    zeros16 = jnp.zeros((16,), jnp.float32)

    def zero_row(i, c):
        for d in range(4):
            zx_v[i, pl.ds(d * 16, 16)] = zeros16
        return c

    lax.fori_loop(0, 128, zero_row, 0)

    onehot = jnp.where(lax.iota(jnp.int32, 16) == 0,
                       jnp.float32(1.0), jnp.float32(0.0))
    zbase = sid * NPT

    def wait_scatter(j):
        pltpu.make_async_copy(bufs.at[0], agg_sh.at[col_v.at[j]],
                              ssem).wait()

    for p, (xt_hbm, out_hbm) in enumerate(
            ((None, aggae_out), (x0_hbm, aggx0_out), (x1_hbm, aggx1_out))):
        for t in range(5):
            pltpu.sync_copy(zx_v, agg_sh.at[pl.ds(zbase + t * 128, 128)])
        plsc.subcore_barrier()

        if p > 0:
            pltpu.async_copy(xt_hbm.at[row_v.at[0]], bufs.at[0], gsem)
            pltpu.async_copy(xt_hbm.at[row_v.at[1]], bufs.at[1], gsem)

        def outer(j5, c, p=p, xt_hbm=xt_hbm):
            for r in range(NBUF):
                j = j5 * NBUF + r
                buf = bufs.at[r]
                pl.when(j >= 3)(lambda: wait_scatter(j - 3))
                if p > 0:
                    def _ahead(j=j, r=r, xt_hbm=xt_hbm):
                        pltpu.async_copy(xt_hbm.at[row_v.at[j + 2]],
                                         bufs.at[(r + 2) % NBUF], gsem)
                    pl.when(j + 2 < NCH)(_ahead)
                    pltpu.make_async_copy(xt_hbm.at[row_v.at[j]], buf,
                                          gsem).wait()
                else:
                    pltpu.sync_copy(ea_hbm.at[wid, j], ea_v)
                    for k in range(CH // 16):
                        ridx = row_v[j, pl.ds(k * 16, 16)]
                        sq = (plsc.load_gather(sqn_v, [ridx])
                              + sqna_v[j, pl.ds(k * 16, 16)])
                        y = _rsqrt16(jnp.maximum(sq, jnp.float32(1e-30)))
                        w = jnp.where(sq > jnp.float32(1e-24), y,
                                      jnp.float32(1e12))
                        w_v[j, pl.ds(k * 16, 16)] = w

                def per_edge(i, cc, buf=buf, p=p):
                    w = w_v[j, pl.ds(i, 16)][0]
                    if p == 0:
                        buf[i, pl.ds(0, 16)] = ea_v[i, :] * w
                        buf[i, pl.ds(16, 16)] = onehot
                        buf[i, pl.ds(32, 16)] = zeros16
                        buf[i, pl.ds(48, 16)] = zeros16
                    else:
                        for d in range(4):
                            buf[i, pl.ds(d * 16, 16)] = \
                                buf[i, pl.ds(d * 16, 16)] * w
                    return cc

                lax.fori_loop(0, CH, per_edge, 0, unroll=4)
                pltpu.async_copy(buf, agg_sh.at[col_v.at[j]], ssem, add=True)
            return c

        lax.fori_loop(0, NCH // NBUF, outer, 0)
        for jt in (NCH - 3, NCH - 2, NCH - 1):
            wait_scatter(jt)
        plsc.subcore_barrier()

        pltpu.sync_copy(agg_sh.at[pl.ds(zbase, NPT)],
                        out_hbm.at[cid, pl.ds(zbase, NPT)])


_SC_MESH = plsc.VectorSubcoreMesh(core_axis_name="c", subcore_axis_name="s")

_sc_aggregate = pl.kernel(
    _sc_aggregate_body,
    out_type=(
        jax.ShapeDtypeStruct((2, N_PAD, D_HALF), jnp.float32),
        jax.ShapeDtypeStruct((2, N_PAD, D_HALF), jnp.float32),
        jax.ShapeDtypeStruct((2, N_PAD, D_HALF), jnp.float32),
    ),
    mesh=_SC_MESH,
    scratch_types=[
        pltpu.VMEM((N_NODES,), jnp.float32),
        pltpu.VMEM((NCH, CH), jnp.int32),
        pltpu.VMEM((NCH, CH), jnp.int32),
        pltpu.VMEM((NCH, CH), jnp.float32),
        pltpu.VMEM((CH, D_EDGE), jnp.float32),
        pltpu.VMEM((NBUF, CH, D_HALF), jnp.float32),
        pltpu.VMEM((NCH, CH + 16), jnp.float32),
        pltpu.VMEM((128, D_HALF), jnp.float32),
        pltpu.VMEM_SHARED((N_PAD, D_HALF), jnp.float32),
        pltpu.SemaphoreType.DMA,
        pltpu.SemaphoreType.DMA,
    ],
    compiler_params=pltpu.CompilerParams(needs_layout_passes=False,
                                         use_tc_tiling_on_sc=False),
    name="sage_sc_aggregate",
)


def _sq_body(x_ref, o_ref):
    v = x_ref[...]
    o_ref[...] = jnp.sum(v * v, axis=1, keepdims=True)


def _row_sqnorm(x, blk):
    rows, d = x.shape
    return pl.pallas_call(
        _sq_body,
        grid=(rows // blk,),
        in_specs=[pl.BlockSpec((blk, d), lambda i: (i, 0))],
        out_specs=pl.BlockSpec((blk, 1), lambda i: (i, 0)),
        out_shape=jax.ShapeDtypeStruct((rows, 1), jnp.float32),
    )(x)


def _post_body(ae_ref, ax0_ref, ax1_ref, wx0_ref, wx1_ref, we_ref, b_ref,
               *out_refs, act, split_out):
    ae = ae_ref[0] + ae_ref[1]
    ax0 = ax0_ref[0] + ax0_ref[1]
    ax1 = ax1_ref[0] + ax1_ref[1]
    cnt = ae[:, 16:17]
    inv = jnp.float32(1.0) / jnp.maximum(cnt, jnp.float32(1.0))
    h = (lax.dot(ax0, wx0_ref[...], preferred_element_type=jnp.float32)
         + lax.dot(ax1, wx1_ref[...], preferred_element_type=jnp.float32)
         + lax.dot(ae[:, :16], we_ref[...], preferred_element_type=jnp.float32))
    h = h * inv + b_ref[...]
    n = jnp.sqrt(jnp.sum(h * h, axis=1, keepdims=True))
    h = act(h / jnp.maximum(n, jnp.float32(1e-12)))
    if split_out:
        o0_ref, o1_ref, sq_ref = out_refs
        o0_ref[...] = h[:, :D_HALF]
        o1_ref[...] = h[:, D_HALF:]
        sq_ref[...] = jnp.sum(h * h, axis=1, keepdims=True)
    else:
        out_refs[0][...] = h


def _post(ae, ax0, ax1, wx0, wx1, we, b, act, split_out, blk=1000):
    body = functools.partial(_post_body, act=act, split_out=split_out)
    if split_out:
        out_specs = [
            pl.BlockSpec((blk, D_HALF), lambda i: (i, 0)),
            pl.BlockSpec((blk, D_HALF), lambda i: (i, 0)),
            pl.BlockSpec((blk, 1), lambda i: (i, 0)),
        ]
        out_shape = [
            jax.ShapeDtypeStruct((N_NODES, D_HALF), jnp.float32),
            jax.ShapeDtypeStruct((N_NODES, D_HALF), jnp.float32),
            jax.ShapeDtypeStruct((N_NODES, 1), jnp.float32),
        ]
    else:
        out_specs = [pl.BlockSpec((blk, D_FEAT), lambda i: (i, 0))]
        out_shape = [jax.ShapeDtypeStruct((N_NODES, D_FEAT), jnp.float32)]
    return pl.pallas_call(
        body,
        grid=(N_NODES // blk,),
        in_specs=[
            pl.BlockSpec((2, blk, D_HALF), lambda i: (0, i, 0)),
            pl.BlockSpec((2, blk, D_HALF), lambda i: (0, i, 0)),
            pl.BlockSpec((2, blk, D_HALF), lambda i: (0, i, 0)),
            pl.BlockSpec((D_HALF, D_FEAT), lambda i: (0, 0)),
            pl.BlockSpec((D_HALF, D_FEAT), lambda i: (0, 0)),
            pl.BlockSpec((D_EDGE, D_FEAT), lambda i: (0, 0)),
            pl.BlockSpec((1, D_FEAT), lambda i: (0, 0)),
        ],
        out_specs=out_specs,
        out_shape=out_shape,
    )(ae, ax0, ax1, wx0, wx1, we, b)


def _layer(x0, x1, sqn, row, col, ea_r, sqna, W, b, act, split_out):
    ae, ax0, ax1 = _sc_aggregate(x0, x1, sqn, row, col, ea_r, sqna)
    return _post(ae, ax0, ax1,
                 W[:D_HALF], W[D_HALF:D_FEAT], W[D_FEAT:],
                 b.reshape(1, D_FEAT), act, split_out)


def kernel(x, edge_index, edge_attr, W1, b1, W2, b2):
    ei = edge_index.astype(jnp.int32)
    row = ei[0].reshape(NW, NCH, CH)
    col = ei[1].reshape(NW, NCH, CH)
    ea_r = edge_attr.reshape(NW, NCH, CH, D_EDGE)

    sqna = _row_sqnorm(edge_attr, blk=8000).reshape(NW, NCH, CH)
    sqnx = _row_sqnorm(x, blk=1000).reshape(N_NODES)

    h0, h1, sqh1 = _layer(x[:, :D_HALF], x[:, D_HALF:], sqnx,
                          row, col, ea_r, sqna, W1, b1,
                          lambda h: jnp.maximum(h, 0.0), split_out=True)
    (out,) = _layer(h0, h1, sqh1.reshape(N_NODES), row, col, ea_r, sqna,
                    W2, b2, lambda h: jax.nn.sigmoid(h), split_out=False)
    return out


# speedup vs baseline: 1.3658x; 1.0121x over previous
import functools

import jax
import jax.numpy as jnp
from jax import lax
from jax.experimental import pallas as pl
from jax.experimental.pallas import tpu as pltpu
from jax.experimental.pallas import tpu_sc as plsc

N_NODES = 10000
N_EDGES = 320000
D_FEAT = 128
D_HALF = 64
D_EDGE = 16

NW = 32
EPW = N_EDGES // NW
CH = 80
NCH = EPW // CH
N_PAD = 10240
NPT = N_PAD // 16
NBUF = 5


def _rsqrt16(a):
    i = plsc.bitcast(a, jnp.int32)
    i = jnp.int32(0x5F3759DF) - (i >> 1)
    y = plsc.bitcast(i, jnp.float32)
    for _ in range(3):
        y = y * (jnp.float32(1.5) - jnp.float32(0.5) * a * y * y)
    return y


def _sc_aggregate_body(x0_hbm, x1_hbm, sqn_hbm, row_hbm, col_hbm, ea_hbm,
                       sqna_hbm,
                       aggae_out, aggx0_out, aggx1_out,
                       sqn_v, row_v, col_v, sqna_v, ea_v, bufs, w_v, zx_v,
                       agg_sh, gsem, ssem):
    cid = lax.axis_index("c")
    sid = lax.axis_index("s")
    wid = cid * 16 + sid

    pltpu.sync_copy(sqn_hbm, sqn_v)
    pltpu.sync_copy(row_hbm.at[wid], row_v)
    pltpu.sync_copy(col_hbm.at[wid], col_v)
    pltpu.sync_copy(sqna_hbm.at[wid], sqna_v)

    zeros16 = jnp.zeros((16,), jnp.float32)

    def zero_row(i, c):
        for d in range(4):
            zx_v[i, pl.ds(d * 16, 16)] = zeros16
        return c

    lax.fori_loop(0, 128, zero_row, 0)

    onehot = jnp.where(lax.iota(jnp.int32, 16) == 0,
                       jnp.float32(1.0), jnp.float32(0.0))
    zbase = sid * NPT

    def wait_scatter(j):
        pltpu.make_async_copy(bufs.at[0], agg_sh.at[col_v.at[j]],
                              ssem).wait()

    for p, (xt_hbm, out_hbm) in enumerate(
            ((None, aggae_out), (x0_hbm, aggx0_out), (x1_hbm, aggx1_out))):
        for t in range(5):
            pltpu.sync_copy(zx_v, agg_sh.at[pl.ds(zbase + t * 128, 128)])
        plsc.subcore_barrier()

        if p > 0:
            pltpu.async_copy(xt_hbm.at[row_v.at[0]], bufs.at[0], gsem)
            pltpu.async_copy(xt_hbm.at[row_v.at[1]], bufs.at[1], gsem)

        def outer(j5, c, p=p, xt_hbm=xt_hbm):
            for r in range(NBUF):
                j = j5 * NBUF + r
                buf = bufs.at[r]
                pl.when(j >= 3)(lambda: wait_scatter(j - 3))
                if p > 0:
                    def _ahead(j=j, r=r, xt_hbm=xt_hbm):
                        pltpu.async_copy(xt_hbm.at[row_v.at[j + 2]],
                                         bufs.at[(r + 2) % NBUF], gsem)
                    pl.when(j + 2 < NCH)(_ahead)
                    pltpu.make_async_copy(xt_hbm.at[row_v.at[j]], buf,
                                          gsem).wait()
                else:
                    pltpu.sync_copy(ea_hbm.at[wid, j], ea_v)
                    for k in range(CH // 16):
                        ridx = row_v[j, pl.ds(k * 16, 16)]
                        sq = (plsc.load_gather(sqn_v, [ridx])
                              + sqna_v[j, pl.ds(k * 16, 16)])
                        y = _rsqrt16(jnp.maximum(sq, jnp.float32(1e-30)))
                        w = jnp.where(sq > jnp.float32(1e-24), y,
                                      jnp.float32(1e12))
                        w_v[j, pl.ds(k * 16, 16)] = w

                def per_edge(i, cc, buf=buf, p=p):
                    w = w_v[j, pl.ds(i, 16)][0]
                    if p == 0:
                        buf[i, pl.ds(0, 16)] = ea_v[i, :] * w
                        buf[i, pl.ds(16, 16)] = onehot
                        buf[i, pl.ds(32, 16)] = zeros16
                        buf[i, pl.ds(48, 16)] = zeros16
                    else:
                        for d in range(4):
                            buf[i, pl.ds(d * 16, 16)] = \
                                buf[i, pl.ds(d * 16, 16)] * w
                    return cc

                lax.fori_loop(0, CH, per_edge, 0, unroll=4)
                pltpu.async_copy(buf, agg_sh.at[col_v.at[j]], ssem, add=True)
            return c

        lax.fori_loop(0, NCH // NBUF, outer, 0)
        for jt in (NCH - 3, NCH - 2, NCH - 1):
            wait_scatter(jt)
        plsc.subcore_barrier()

        pltpu.sync_copy(agg_sh.at[pl.ds(zbase, NPT)],
                        out_hbm.at[cid, pl.ds(zbase, NPT)])


_SC_MESH = plsc.VectorSubcoreMesh(core_axis_name="c", subcore_axis_name="s")

_sc_aggregate = pl.kernel(
    _sc_aggregate_body,
    out_type=(
        jax.ShapeDtypeStruct((2, N_PAD, D_HALF), jnp.float32),
        jax.ShapeDtypeStruct((2, N_PAD, D_HALF), jnp.float32),
        jax.ShapeDtypeStruct((2, N_PAD, D_HALF), jnp.float32),
    ),
    mesh=_SC_MESH,
    scratch_types=[
        pltpu.VMEM((N_NODES,), jnp.float32),
        pltpu.VMEM((NCH, CH), jnp.int32),
        pltpu.VMEM((NCH, CH), jnp.int32),
        pltpu.VMEM((NCH, CH), jnp.float32),
        pltpu.VMEM((CH, D_EDGE), jnp.float32),
        pltpu.VMEM((NBUF, CH, D_HALF), jnp.float32),
        pltpu.VMEM((NCH, CH + 16), jnp.float32),
        pltpu.VMEM((128, D_HALF), jnp.float32),
        pltpu.VMEM_SHARED((N_PAD, D_HALF), jnp.float32),
        pltpu.SemaphoreType.DMA,
        pltpu.SemaphoreType.DMA,
    ],
    compiler_params=pltpu.CompilerParams(needs_layout_passes=False,
                                         use_tc_tiling_on_sc=False),
    name="sage_sc_aggregate",
)


def _sq_body(x_ref, o_ref):
    v = x_ref[...]
    o_ref[...] = jnp.sum(v * v, axis=1, keepdims=True)


def _row_sqnorm(x, blk):
    rows, d = x.shape
    return pl.pallas_call(
        _sq_body,
        grid=(rows // blk,),
        in_specs=[pl.BlockSpec((blk, d), lambda i: (i, 0))],
        out_specs=pl.BlockSpec((blk, 1), lambda i: (i, 0)),
        out_shape=jax.ShapeDtypeStruct((rows, 1), jnp.float32),
    )(x)


def _post_body(ae_ref, ax0_ref, ax1_ref, wx0_ref, wx1_ref, we_ref, b_ref,
               *out_refs, act, split_out):
    ae = ae_ref[0] + ae_ref[1]
    ax0 = ax0_ref[0] + ax0_ref[1]
    ax1 = ax1_ref[0] + ax1_ref[1]
    cnt = ae[:, 16:17]
    inv = jnp.float32(1.0) / jnp.maximum(cnt, jnp.float32(1.0))
    h = (lax.dot(ax0, wx0_ref[...], preferred_element_type=jnp.float32)
         + lax.dot(ax1, wx1_ref[...], preferred_element_type=jnp.float32)
         + lax.dot(ae[:, :16], we_ref[...], preferred_element_type=jnp.float32))
    h = h * inv + b_ref[...]
    n = jnp.sqrt(jnp.sum(h * h, axis=1, keepdims=True))
    h = act(h / jnp.maximum(n, jnp.float32(1e-12)))
    if split_out:
        o0_ref, o1_ref, sq_ref = out_refs
        o0_ref[...] = h[:, :D_HALF]
        o1_ref[...] = h[:, D_HALF:]
        sq_ref[...] = jnp.sum(h * h, axis=1, keepdims=True)
    else:
        out_refs[0][...] = h


def _post(ae, ax0, ax1, wx0, wx1, we, b, act, split_out, blk=2000):
    body = functools.partial(_post_body, act=act, split_out=split_out)
    if split_out:
        out_specs = [
            pl.BlockSpec((blk, D_HALF), lambda i: (i, 0)),
            pl.BlockSpec((blk, D_HALF), lambda i: (i, 0)),
            pl.BlockSpec((blk, 1), lambda i: (i, 0)),
        ]
        out_shape = [
            jax.ShapeDtypeStruct((N_NODES, D_HALF), jnp.float32),
            jax.ShapeDtypeStruct((N_NODES, D_HALF), jnp.float32),
            jax.ShapeDtypeStruct((N_NODES, 1), jnp.float32),
        ]
    else:
        out_specs = [pl.BlockSpec((blk, D_FEAT), lambda i: (i, 0))]
        out_shape = [jax.ShapeDtypeStruct((N_NODES, D_FEAT), jnp.float32)]
    return pl.pallas_call(
        body,
        grid=(N_NODES // blk,),
        in_specs=[
            pl.BlockSpec((2, blk, D_HALF), lambda i: (0, i, 0)),
            pl.BlockSpec((2, blk, D_HALF), lambda i: (0, i, 0)),
            pl.BlockSpec((2, blk, D_HALF), lambda i: (0, i, 0)),
            pl.BlockSpec((D_HALF, D_FEAT), lambda i: (0, 0)),
            pl.BlockSpec((D_HALF, D_FEAT), lambda i: (0, 0)),
            pl.BlockSpec((D_EDGE, D_FEAT), lambda i: (0, 0)),
            pl.BlockSpec((1, D_FEAT), lambda i: (0, 0)),
        ],
        out_specs=out_specs,
        out_shape=out_shape,
    )(ae, ax0, ax1, wx0, wx1, we, b)


def _layer(x0, x1, sqn, row, col, ea_r, sqna, W, b, act, split_out):
    ae, ax0, ax1 = _sc_aggregate(x0, x1, sqn, row, col, ea_r, sqna)
    return _post(ae, ax0, ax1,
                 W[:D_HALF], W[D_HALF:D_FEAT], W[D_FEAT:],
                 b.reshape(1, D_FEAT), act, split_out)


def kernel(x, edge_index, edge_attr, W1, b1, W2, b2):
    ei = edge_index.astype(jnp.int32)
    row = ei[0].reshape(NW, NCH, CH)
    col = ei[1].reshape(NW, NCH, CH)
    ea_r = edge_attr.reshape(NW, NCH, CH, D_EDGE)

    sqna = _row_sqnorm(edge_attr, blk=8000).reshape(NW, NCH, CH)
    sqnx = _row_sqnorm(x, blk=2000).reshape(N_NODES)

    h0, h1, sqh1 = _layer(x[:, :D_HALF], x[:, D_HALF:], sqnx,
                          row, col, ea_r, sqna, W1, b1,
                          lambda h: jnp.maximum(h, 0.0), split_out=True)
    (out,) = _layer(h0, h1, sqh1.reshape(N_NODES), row, col, ea_r, sqna,
                    W2, b2, lambda h: jax.nn.sigmoid(h), split_out=False)
    return out


# speedup vs baseline: 1.5561x; 1.1394x over previous
import functools

import jax
import jax.numpy as jnp
from jax import lax
from jax.experimental import pallas as pl
from jax.experimental.pallas import tpu as pltpu
from jax.experimental.pallas import tpu_sc as plsc

N_NODES = 10000
N_EDGES = 320000
D_FEAT = 128
D_HALF = 64
D_EDGE = 16

NW = 32
EPW = N_EDGES // NW
CH = 80
NCH = EPW // CH
N_PAD = 10240
NPT = N_PAD // 16
NBUF = 5


def _rsqrt16(a):
    i = plsc.bitcast(a, jnp.int32)
    i = jnp.int32(0x5F3759DF) - (i >> 1)
    y = plsc.bitcast(i, jnp.float32)
    for _ in range(3):
        y = y * (jnp.float32(1.5) - jnp.float32(0.5) * a * y * y)
    return y


def _sc_aggregate_body(x0_hbm, x1_hbm, sqn_hbm, row_hbm, col_hbm, ea_hbm,
                       sqna_hbm,
                       aggae_out, aggx0_out, aggx1_out,
                       sqn_v, row_v, col_v, sqna_v, ea_v, bufs, w_v, zx_v,
                       agg_sh, gsem, ssem, esem):
    cid = lax.axis_index("c")
    sid = lax.axis_index("s")
    wid = cid * 16 + sid

    pltpu.sync_copy(sqn_hbm, sqn_v)
    pltpu.sync_copy(row_hbm.at[wid], row_v)
    pltpu.sync_copy(col_hbm.at[wid], col_v)
    pltpu.sync_copy(sqna_hbm.at[wid], sqna_v)

    zeros16 = jnp.zeros((16,), jnp.float32)

    def zero_row(i, c):
        for d in range(4):
            zx_v[i, pl.ds(d * 16, 16)] = zeros16
        return c

    lax.fori_loop(0, 128, zero_row, 0)

    onehot = jnp.where(lax.iota(jnp.int32, 16) == 0,
                       jnp.float32(1.0), jnp.float32(0.0))
    zbase = sid * NPT

    def wait_scatter(j):
        pltpu.make_async_copy(bufs.at[0], agg_sh.at[col_v.at[j]],
                              ssem).wait()

    for p, (xt_hbm, out_hbm) in enumerate(
            ((None, aggae_out), (x0_hbm, aggx0_out), (x1_hbm, aggx1_out))):
        for t in range(5):
            pltpu.sync_copy(zx_v, agg_sh.at[pl.ds(zbase + t * 128, 128)])
        plsc.subcore_barrier()

        if p > 0:
            pltpu.async_copy(xt_hbm.at[row_v.at[0]], bufs.at[0], gsem)
            pltpu.async_copy(xt_hbm.at[row_v.at[1]], bufs.at[1], gsem)
        else:
            pltpu.sync_copy(ea_hbm.at[wid, 0], ea_v.at[0])

        def outer(j5, c, p=p, xt_hbm=xt_hbm):
            for r in range(NBUF):
                j = j5 * NBUF + r
                buf = bufs.at[r]
                pl.when(j >= 3)(lambda: wait_scatter(j - 3))
                if p > 0:
                    def _ahead(j=j, r=r, xt_hbm=xt_hbm):
                        pltpu.async_copy(xt_hbm.at[row_v.at[j + 2]],
                                         bufs.at[(r + 2) % NBUF], gsem)
                    pl.when(j + 2 < NCH)(_ahead)
                    pltpu.make_async_copy(xt_hbm.at[row_v.at[j]], buf,
                                          gsem).wait()
                else:
                    def _ea_ahead(j=j):
                        pltpu.async_copy(ea_hbm.at[wid, j + 1],
                                         ea_v.at[(j + 1) % 2], esem)
                    pl.when(j + 1 < NCH)(_ea_ahead)

                    def _ea_wait():
                        pltpu.make_async_copy(ea_hbm.at[wid, 0],
                                              ea_v.at[0], esem).wait()
                    pl.when(j >= 1)(_ea_wait)
                    for k in range(CH // 16):
                        ridx = row_v[j, pl.ds(k * 16, 16)]
                        sq = (plsc.load_gather(sqn_v, [ridx])
                              + sqna_v[j, pl.ds(k * 16, 16)])
                        y = _rsqrt16(jnp.maximum(sq, jnp.float32(1e-30)))
                        w = jnp.where(sq > jnp.float32(1e-24), y,
                                      jnp.float32(1e12))
                        w_v[j, pl.ds(k * 16, 16)] = w

                def per_edge(i, cc, buf=buf, p=p):
                    w = w_v[j, pl.ds(i, 16)][0]
                    if p == 0:
                        buf[i, pl.ds(0, 16)] = ea_v[j % 2, i, :] * w
                        buf[i, pl.ds(16, 16)] = onehot
                        buf[i, pl.ds(32, 16)] = zeros16
                        buf[i, pl.ds(48, 16)] = zeros16
                    else:
                        for d in range(4):
                            buf[i, pl.ds(d * 16, 16)] = \
                                buf[i, pl.ds(d * 16, 16)] * w
                    return cc

                lax.fori_loop(0, CH, per_edge, 0, unroll=4)
                pltpu.async_copy(buf, agg_sh.at[col_v.at[j]], ssem, add=True)
            return c

        lax.fori_loop(0, NCH // NBUF, outer, 0)
        for jt in (NCH - 3, NCH - 2, NCH - 1):
            wait_scatter(jt)
        plsc.subcore_barrier()

        pltpu.sync_copy(agg_sh.at[pl.ds(zbase, NPT)],
                        out_hbm.at[cid, pl.ds(zbase, NPT)])


_SC_MESH = plsc.VectorSubcoreMesh(core_axis_name="c", subcore_axis_name="s")

_sc_aggregate = pl.kernel(
    _sc_aggregate_body,
    out_type=(
        jax.ShapeDtypeStruct((2, N_PAD, D_HALF), jnp.float32),
        jax.ShapeDtypeStruct((2, N_PAD, D_HALF), jnp.float32),
        jax.ShapeDtypeStruct((2, N_PAD, D_HALF), jnp.float32),
    ),
    mesh=_SC_MESH,
    scratch_types=[
        pltpu.VMEM((N_NODES,), jnp.float32),
        pltpu.VMEM((NCH, CH), jnp.int32),
        pltpu.VMEM((NCH, CH), jnp.int32),
        pltpu.VMEM((NCH, CH), jnp.float32),
        pltpu.VMEM((2, CH, D_EDGE), jnp.float32),
        pltpu.VMEM((NBUF, CH, D_HALF), jnp.float32),
        pltpu.VMEM((NCH, CH + 16), jnp.float32),
        pltpu.VMEM((128, D_HALF), jnp.float32),
        pltpu.VMEM_SHARED((N_PAD, D_HALF), jnp.float32),
        pltpu.SemaphoreType.DMA,
        pltpu.SemaphoreType.DMA,
        pltpu.SemaphoreType.DMA,
    ],
    compiler_params=pltpu.CompilerParams(needs_layout_passes=False,
                                         use_tc_tiling_on_sc=False),
    name="sage_sc_aggregate",
)


def _sq_body(x_ref, o_ref):
    v = x_ref[...]
    o_ref[...] = jnp.sum(v * v, axis=1, keepdims=True)


def _row_sqnorm(x, blk):
    rows, d = x.shape
    return pl.pallas_call(
        _sq_body,
        grid=(rows // blk,),
        in_specs=[pl.BlockSpec((blk, d), lambda i: (i, 0))],
        out_specs=pl.BlockSpec((blk, 1), lambda i: (i, 0)),
        out_shape=jax.ShapeDtypeStruct((rows, 1), jnp.float32),
    )(x)


def _post_body(ae_ref, ax0_ref, ax1_ref, wx0_ref, wx1_ref, we_ref, b_ref,
               *out_refs, act, split_out):
    ae = ae_ref[0] + ae_ref[1]
    ax0 = ax0_ref[0] + ax0_ref[1]
    ax1 = ax1_ref[0] + ax1_ref[1]
    cnt = ae[:, 16:17]
    inv = jnp.float32(1.0) / jnp.maximum(cnt, jnp.float32(1.0))
    h = (lax.dot(ax0, wx0_ref[...], preferred_element_type=jnp.float32)
         + lax.dot(ax1, wx1_ref[...], preferred_element_type=jnp.float32)
         + lax.dot(ae[:, :16], we_ref[...], preferred_element_type=jnp.float32))
    h = h * inv + b_ref[...]
    n = jnp.sqrt(jnp.sum(h * h, axis=1, keepdims=True))
    h = act(h / jnp.maximum(n, jnp.float32(1e-12)))
    if split_out:
        o0_ref, o1_ref, sq_ref = out_refs
        o0_ref[...] = h[:, :D_HALF]
        o1_ref[...] = h[:, D_HALF:]
        sq_ref[...] = jnp.sum(h * h, axis=1, keepdims=True)
    else:
        out_refs[0][...] = h


def _post(ae, ax0, ax1, wx0, wx1, we, b, act, split_out, blk=2000):
    body = functools.partial(_post_body, act=act, split_out=split_out)
    if split_out:
        out_specs = [
            pl.BlockSpec((blk, D_HALF), lambda i: (i, 0)),
            pl.BlockSpec((blk, D_HALF), lambda i: (i, 0)),
            pl.BlockSpec((blk, 1), lambda i: (i, 0)),
        ]
        out_shape = [
            jax.ShapeDtypeStruct((N_NODES, D_HALF), jnp.float32),
            jax.ShapeDtypeStruct((N_NODES, D_HALF), jnp.float32),
            jax.ShapeDtypeStruct((N_NODES, 1), jnp.float32),
        ]
    else:
        out_specs = [pl.BlockSpec((blk, D_FEAT), lambda i: (i, 0))]
        out_shape = [jax.ShapeDtypeStruct((N_NODES, D_FEAT), jnp.float32)]
    return pl.pallas_call(
        body,
        grid=(N_NODES // blk,),
        in_specs=[
            pl.BlockSpec((2, blk, D_HALF), lambda i: (0, i, 0)),
            pl.BlockSpec((2, blk, D_HALF), lambda i: (0, i, 0)),
            pl.BlockSpec((2, blk, D_HALF), lambda i: (0, i, 0)),
            pl.BlockSpec((D_HALF, D_FEAT), lambda i: (0, 0)),
            pl.BlockSpec((D_HALF, D_FEAT), lambda i: (0, 0)),
            pl.BlockSpec((D_EDGE, D_FEAT), lambda i: (0, 0)),
            pl.BlockSpec((1, D_FEAT), lambda i: (0, 0)),
        ],
        out_specs=out_specs,
        out_shape=out_shape,
    )(ae, ax0, ax1, wx0, wx1, we, b)


def _layer(x0, x1, sqn, row, col, ea_r, sqna, W, b, act, split_out):
    ae, ax0, ax1 = _sc_aggregate(x0, x1, sqn, row, col, ea_r, sqna)
    return _post(ae, ax0, ax1,
                 W[:D_HALF], W[D_HALF:D_FEAT], W[D_FEAT:],
                 b.reshape(1, D_FEAT), act, split_out)


def kernel(x, edge_index, edge_attr, W1, b1, W2, b2):
    ei = edge_index.astype(jnp.int32)
    row = ei[0].reshape(NW, NCH, CH)
    col = ei[1].reshape(NW, NCH, CH)
    ea_r = edge_attr.reshape(NW, NCH, CH, D_EDGE)

    sqna = _row_sqnorm(edge_attr, blk=8000).reshape(NW, NCH, CH)
    sqnx = _row_sqnorm(x, blk=2000).reshape(N_NODES)

    h0, h1, sqh1 = _layer(x[:, :D_HALF], x[:, D_HALF:], sqnx,
                          row, col, ea_r, sqna, W1, b1,
                          lambda h: jnp.maximum(h, 0.0), split_out=True)
    (out,) = _layer(h0, h1, sqh1.reshape(N_NODES), row, col, ea_r, sqna,
                    W2, b2, lambda h: jax.nn.sigmoid(h), split_out=False)
    return out


# speedup vs baseline: 1.5801x; 1.0154x over previous
import functools

import jax
import jax.numpy as jnp
from jax import lax
from jax.experimental import pallas as pl
from jax.experimental.pallas import tpu as pltpu
from jax.experimental.pallas import tpu_sc as plsc

N_NODES = 10000
N_EDGES = 320000
D_FEAT = 128
D_HALF = 64
D_EDGE = 16

NW = 32
EPW = N_EDGES // NW
CH = 80
NCH = EPW // CH
N_PAD = 10240
NPT = N_PAD // 16
NBUF = 5


def _rsqrt16(a):
    i = plsc.bitcast(a, jnp.int32)
    i = jnp.int32(0x5F3759DF) - (i >> 1)
    y = plsc.bitcast(i, jnp.float32)
    for _ in range(3):
        y = y * (jnp.float32(1.5) - jnp.float32(0.5) * a * y * y)
    return y


def _sc_aggregate_body(x0_hbm, x1_hbm, sqn_hbm, row_hbm, col_hbm, ea_hbm,
                       sqna_hbm,
                       aggae_out, aggx0_out, aggx1_out,
                       sqn_v, row_v, col_v, sqna_v, ea_v, bufs, w_v, zx_v,
                       agg_sh, gsem, ssem, esem):
    cid = lax.axis_index("c")
    sid = lax.axis_index("s")
    wid = cid * 16 + sid

    pltpu.sync_copy(sqn_hbm, sqn_v)
    pltpu.sync_copy(row_hbm.at[wid], row_v)
    pltpu.sync_copy(col_hbm.at[wid], col_v)
    pltpu.sync_copy(sqna_hbm.at[wid], sqna_v)

    zeros16 = jnp.zeros((16,), jnp.float32)

    def zero_row(i, c):
        for d in range(4):
            zx_v[i, pl.ds(d * 16, 16)] = zeros16
        return c

    lax.fori_loop(0, 128, zero_row, 0)

    onehot = jnp.where(lax.iota(jnp.int32, 16) == 0,
                       jnp.float32(1.0), jnp.float32(0.0))
    zbase = sid * NPT

    def wait_scatter(j):
        pltpu.make_async_copy(bufs.at[0], agg_sh.at[col_v.at[j]],
                              ssem).wait()

    for p, (xt_hbm, out_hbm) in enumerate(
            ((None, aggae_out), (x0_hbm, aggx0_out), (x1_hbm, aggx1_out))):
        for t in range(5):
            pltpu.sync_copy(zx_v, agg_sh.at[pl.ds(zbase + t * 128, 128)])
        plsc.subcore_barrier()

        if p > 0:
            pltpu.async_copy(xt_hbm.at[row_v.at[0]], bufs.at[0], gsem)
            pltpu.async_copy(xt_hbm.at[row_v.at[1]], bufs.at[1], gsem)
        else:
            pltpu.sync_copy(ea_hbm.at[wid, 0], ea_v.at[0])

        def outer(j5, c, p=p, xt_hbm=xt_hbm):
            for r in range(NBUF):
                j = j5 * NBUF + r
                buf = bufs.at[r]
                pl.when(j >= 3)(lambda: wait_scatter(j - 3))
                if p > 0:
                    def _ahead(j=j, r=r, xt_hbm=xt_hbm):
                        pltpu.async_copy(xt_hbm.at[row_v.at[j + 2]],
                                         bufs.at[(r + 2) % NBUF], gsem)
                    pl.when(j + 2 < NCH)(_ahead)
                    pltpu.make_async_copy(xt_hbm.at[row_v.at[j]], buf,
                                          gsem).wait()
                else:
                    def _ea_ahead(j=j):
                        pltpu.async_copy(ea_hbm.at[wid, j + 1],
                                         ea_v.at[(j + 1) % 2], esem)
                    pl.when(j + 1 < NCH)(_ea_ahead)

                    def _ea_wait():
                        pltpu.make_async_copy(ea_hbm.at[wid, 0],
                                              ea_v.at[0], esem).wait()
                    pl.when(j >= 1)(_ea_wait)
                    for k in range(CH // 16):
                        ridx = row_v[j, pl.ds(k * 16, 16)]
                        sq = (plsc.load_gather(sqn_v, [ridx])
                              + sqna_v[j, pl.ds(k * 16, 16)])
                        y = _rsqrt16(jnp.maximum(sq, jnp.float32(1e-30)))
                        w = jnp.where(sq > jnp.float32(1e-24), y,
                                      jnp.float32(1e12))
                        w_v[j, pl.ds(k * 16, 16)] = w

                def per_edge(i, cc, buf=buf, p=p):
                    w = w_v[j, pl.ds(i, 16)][0]
                    if p == 0:
                        buf[i, pl.ds(0, 16)] = ea_v[j % 2, i, :] * w
                        buf[i, pl.ds(16, 16)] = onehot
                        buf[i, pl.ds(32, 16)] = zeros16
                        buf[i, pl.ds(48, 16)] = zeros16
                    else:
                        for d in range(4):
                            buf[i, pl.ds(d * 16, 16)] = \
                                buf[i, pl.ds(d * 16, 16)] * w
                    return cc

                lax.fori_loop(0, CH, per_edge, 0, unroll=4)
                pltpu.async_copy(buf, agg_sh.at[col_v.at[j]], ssem, add=True)
            return c

        lax.fori_loop(0, NCH // NBUF, outer, 0)
        for jt in (NCH - 3, NCH - 2, NCH - 1):
            wait_scatter(jt)
        plsc.subcore_barrier()

        pltpu.sync_copy(agg_sh.at[pl.ds(zbase, NPT)],
                        out_hbm.at[cid, pl.ds(zbase, NPT)])


_SC_MESH = plsc.VectorSubcoreMesh(core_axis_name="c", subcore_axis_name="s")

_sc_aggregate = pl.kernel(
    _sc_aggregate_body,
    out_type=(
        jax.ShapeDtypeStruct((2, N_PAD, D_HALF), jnp.float32),
        jax.ShapeDtypeStruct((2, N_PAD, D_HALF), jnp.float32),
        jax.ShapeDtypeStruct((2, N_PAD, D_HALF), jnp.float32),
    ),
    mesh=_SC_MESH,
    scratch_types=[
        pltpu.VMEM((N_NODES,), jnp.float32),
        pltpu.VMEM((NCH, CH), jnp.int32),
        pltpu.VMEM((NCH, CH), jnp.int32),
        pltpu.VMEM((NCH, CH), jnp.float32),
        pltpu.VMEM((2, CH, D_EDGE), jnp.float32),
        pltpu.VMEM((NBUF, CH, D_HALF), jnp.float32),
        pltpu.VMEM((NCH, CH + 16), jnp.float32),
        pltpu.VMEM((128, D_HALF), jnp.float32),
        pltpu.VMEM_SHARED((N_PAD, D_HALF), jnp.float32),
        pltpu.SemaphoreType.DMA,
        pltpu.SemaphoreType.DMA,
        pltpu.SemaphoreType.DMA,
    ],
    compiler_params=pltpu.CompilerParams(needs_layout_passes=False,
                                         use_tc_tiling_on_sc=False),
    name="sage_sc_aggregate",
)


def _sq_body(x_ref, o_ref):
    v = x_ref[...]
    o_ref[...] = jnp.sum(v * v, axis=1, keepdims=True)


def _row_sqnorm(x, blk):
    rows, d = x.shape
    return pl.pallas_call(
        _sq_body,
        grid=(rows // blk,),
        in_specs=[pl.BlockSpec((blk, d), lambda i: (i, 0))],
        out_specs=pl.BlockSpec((blk, 1), lambda i: (i, 0)),
        out_shape=jax.ShapeDtypeStruct((rows, 1), jnp.float32),
    )(x)


def _prep_body(ea_ref, x_ref, oea_ref, ox_ref):
    v = ea_ref[...]
    oea_ref[...] = jnp.sum(v * v, axis=1, keepdims=True)
    u = x_ref[...]
    ox_ref[...] = jnp.sum(u * u, axis=1, keepdims=True)


def _prep_sqnorms(ea, x, grid=25):
    eb = N_EDGES // grid
    xb = N_NODES // grid
    return pl.pallas_call(
        _prep_body,
        grid=(grid,),
        in_specs=[
            pl.BlockSpec((eb, D_EDGE), lambda i: (i, 0)),
            pl.BlockSpec((xb, D_FEAT), lambda i: (i, 0)),
        ],
        out_specs=[
            pl.BlockSpec((eb, 1), lambda i: (i, 0)),
            pl.BlockSpec((xb, 1), lambda i: (i, 0)),
        ],
        out_shape=[
            jax.ShapeDtypeStruct((N_EDGES, 1), jnp.float32),
            jax.ShapeDtypeStruct((N_NODES, 1), jnp.float32),
        ],
    )(ea, x)


def _post_body(ae_ref, ax0_ref, ax1_ref, wx0_ref, wx1_ref, we_ref, b_ref,
               *out_refs, act, split_out):
    ae = ae_ref[0] + ae_ref[1]
    ax0 = ax0_ref[0] + ax0_ref[1]
    ax1 = ax1_ref[0] + ax1_ref[1]
    cnt = ae[:, 16:17]
    inv = jnp.float32(1.0) / jnp.maximum(cnt, jnp.float32(1.0))
    h = (lax.dot(ax0, wx0_ref[...], preferred_element_type=jnp.float32)
         + lax.dot(ax1, wx1_ref[...], preferred_element_type=jnp.float32)
         + lax.dot(ae[:, :16], we_ref[...], preferred_element_type=jnp.float32))
    h = h * inv + b_ref[...]
    n = jnp.sqrt(jnp.sum(h * h, axis=1, keepdims=True))
    h = act(h / jnp.maximum(n, jnp.float32(1e-12)))
    if split_out:
        o0_ref, o1_ref, sq_ref = out_refs
        o0_ref[...] = h[:, :D_HALF]
        o1_ref[...] = h[:, D_HALF:]
        sq_ref[...] = jnp.sum(h * h, axis=1, keepdims=True)
    else:
        out_refs[0][...] = h


def _post(ae, ax0, ax1, wx0, wx1, we, b, act, split_out, blk=2000):
    body = functools.partial(_post_body, act=act, split_out=split_out)
    if split_out:
        out_specs = [
            pl.BlockSpec((blk, D_HALF), lambda i: (i, 0)),
            pl.BlockSpec((blk, D_HALF), lambda i: (i, 0)),
            pl.BlockSpec((blk, 1), lambda i: (i, 0)),
        ]
        out_shape = [
            jax.ShapeDtypeStruct((N_NODES, D_HALF), jnp.float32),
            jax.ShapeDtypeStruct((N_NODES, D_HALF), jnp.float32),
            jax.ShapeDtypeStruct((N_NODES, 1), jnp.float32),
        ]
    else:
        out_specs = [pl.BlockSpec((blk, D_FEAT), lambda i: (i, 0))]
        out_shape = [jax.ShapeDtypeStruct((N_NODES, D_FEAT), jnp.float32)]
    return pl.pallas_call(
        body,
        grid=(N_NODES // blk,),
        in_specs=[
            pl.BlockSpec((2, blk, D_HALF), lambda i: (0, i, 0)),
            pl.BlockSpec((2, blk, D_HALF), lambda i: (0, i, 0)),
            pl.BlockSpec((2, blk, D_HALF), lambda i: (0, i, 0)),
            pl.BlockSpec((D_HALF, D_FEAT), lambda i: (0, 0)),
            pl.BlockSpec((D_HALF, D_FEAT), lambda i: (0, 0)),
            pl.BlockSpec((D_EDGE, D_FEAT), lambda i: (0, 0)),
            pl.BlockSpec((1, D_FEAT), lambda i: (0, 0)),
        ],
        out_specs=out_specs,
        out_shape=out_shape,
    )(ae, ax0, ax1, wx0, wx1, we, b)


def _layer(x0, x1, sqn, row, col, ea_r, sqna, W, b, act, split_out):
    ae, ax0, ax1 = _sc_aggregate(x0, x1, sqn, row, col, ea_r, sqna)
    return _post(ae, ax0, ax1,
                 W[:D_HALF], W[D_HALF:D_FEAT], W[D_FEAT:],
                 b.reshape(1, D_FEAT), act, split_out)


def kernel(x, edge_index, edge_attr, W1, b1, W2, b2):
    ei = edge_index.astype(jnp.int32)
    row = ei[0].reshape(NW, NCH, CH)
    col = ei[1].reshape(NW, NCH, CH)
    ea_r = edge_attr.reshape(NW, NCH, CH, D_EDGE)

    sqna_2d, sqnx_2d = _prep_sqnorms(edge_attr, x)
    sqna = sqna_2d.reshape(NW, NCH, CH)
    sqnx = sqnx_2d.reshape(N_NODES)

    h0, h1, sqh1 = _layer(x[:, :D_HALF], x[:, D_HALF:], sqnx,
                          row, col, ea_r, sqna, W1, b1,
                          lambda h: jnp.maximum(h, 0.0), split_out=True)
    (out,) = _layer(h0, h1, sqh1.reshape(N_NODES), row, col, ea_r, sqna,
                    W2, b2, lambda h: jax.nn.sigmoid(h), split_out=False)
    return out
